# speedup vs baseline: 63.2123x; 63.2123x over previous
import functools

import jax
import jax.numpy as jnp
from jax import lax
from jax.experimental import pallas as pl
from jax.experimental.pallas import tpu as pltpu
from jax.experimental.pallas import tpu_sc as plsc

B, N, D = 8, 1250, 128
E = 40000
L = 4
Np = 1280
NC, NS = 2, 16
EP = E // NS
ROWS = 20
EPP = ROWS * 128
STRIPE = (Np * Np) // NS
ZCH = STRIPE // 8
GPC = B // NC


def _sc_body(dst_hbm, src_hbm, vals_hbm, out_hbm,
             shared, srcv, dstv, idxb, valb, zbuf):
  c = lax.axis_index("c")
  s = lax.axis_index("s")

  pltpu.sync_copy(vals_hbm, valb)

  def _z(i, _):
    zbuf[pl.ds(i * 16, 16)] = jnp.zeros((16,), jnp.float32)
    return 0
  lax.fori_loop(0, ZCH // 16, _z, 0)

  soff = pl.multiple_of(s * STRIPE, 256)

  def _graph(r, _):
    b = r * NC + c
    for q in range(8):
      pltpu.sync_copy(zbuf, shared.at[pl.ds(soff + q * ZCH, ZCH)])
    plsc.subcore_barrier()
    pltpu.sync_copy(src_hbm.at[b, s], srcv)
    pltpu.sync_copy(dst_hbm.at[b, s], dstv)
    for j in range(ROWS):
      for k in range(8):
        sl = pl.ds(k * 16, 16)
        idxb[j, sl] = dstv[j, sl] * Np + srcv[j, sl]
      pltpu.sync_copy(valb.at[j], shared.at[idxb.at[j]], add=True)
    plsc.subcore_barrier()
    pltpu.sync_copy(shared.at[pl.ds(soff, STRIPE)],
                    out_hbm.at[b, pl.ds(soff, STRIPE)])
    plsc.subcore_barrier()
    return 0

  lax.fori_loop(0, GPC, _graph, 0)


def _build_adjacency(dst4, src4, vals):
  mesh = plsc.VectorSubcoreMesh(core_axis_name="c", subcore_axis_name="s")
  f = pl.kernel(
      _sc_body,
      out_type=jax.ShapeDtypeStruct((B, Np * Np), jnp.float32),
      mesh=mesh,
      scratch_types=[
          pltpu.VMEM_SHARED((Np * Np,), jnp.float32),
          pltpu.VMEM((ROWS, 128), jnp.int32),
          pltpu.VMEM((ROWS, 128), jnp.int32),
          pltpu.VMEM((ROWS, 128), jnp.int32),
          pltpu.VMEM((ROWS, 128), jnp.float32),
          pltpu.VMEM((ZCH,), jnp.float32),
      ],
  )
  return f(dst4, src4, vals)


def _tc_body(a_ref, x_ref, w_ref, b_ref, o_ref, an_ref):
  a = a_ref[0]
  deg = jnp.sum(a, axis=1, keepdims=True) + 1.0
  dis = lax.rsqrt(deg)
  rows = lax.broadcasted_iota(jnp.int32, (Np, Np), 0)
  cols = lax.broadcasted_iota(jnp.int32, (Np, Np), 1)
  eye = jnp.where(rows == cols, 1.0, 0.0).astype(jnp.float32)
  an_ref[...] = (a + eye) * dis * jnp.reshape(dis, (1, Np))
  x = x_ref[0]
  for l in range(L):
    h = jnp.dot(x, w_ref[l], preferred_element_type=jnp.float32)
    y = jnp.dot(an_ref[...], h, preferred_element_type=jnp.float32)
    x = jnp.tanh(y + b_ref[l][None, :])
  o_ref[0] = x


def _gcn_stack(adj, x_pad, wst, bst):
  return pl.pallas_call(
      _tc_body,
      grid=(B,),
      in_specs=[
          pl.BlockSpec((1, Np, Np), lambda b: (b, 0, 0)),
          pl.BlockSpec((1, Np, D), lambda b: (b, 0, 0)),
          pl.BlockSpec((L, D, D), lambda b: (0, 0, 0)),
          pl.BlockSpec((L, D), lambda b: (0, 0)),
      ],
      out_specs=pl.BlockSpec((1, Np, D), lambda b: (b, 0, 0)),
      out_shape=jax.ShapeDtypeStruct((B, Np, D), jnp.float32),
      scratch_shapes=[pltpu.VMEM((Np, Np), jnp.float32)],
  )(adj, x_pad, wst, bst)


@jax.jit
def kernel(batch_node_tsr, edge_tsr_list, batch_last_node_idx_list,
           W0, b0, W1, b1, W2, b2, W3, b3):
  del batch_last_node_idx_list
  src = edge_tsr_list[:, 0, :].reshape(B, NS, EP)
  dst = edge_tsr_list[:, 1, :].reshape(B, NS, EP)
  pad = ((0, 0), (0, 0), (0, EPP - EP))
  src4 = jnp.pad(src, pad).reshape(B, NS, ROWS, 128)
  dst4 = jnp.pad(dst, pad).reshape(B, NS, ROWS, 128)
  vals = (jnp.arange(EPP) < EP).astype(jnp.float32).reshape(ROWS, 128)

  adj = _build_adjacency(dst4, src4, vals).reshape(B, Np, Np)

  x_pad = jnp.pad(batch_node_tsr, ((0, 0), (0, Np - N), (0, 0)))
  wst = jnp.stack([W0, W1, W2, W3])
  bst = jnp.stack([b0, b1, b2, b3])
  out = _gcn_stack(adj, x_pad, wst, bst)
  return out[:, :N, :]


# speedup vs baseline: 82.5573x; 1.3060x over previous
import jax
import jax.numpy as jnp
from jax import lax
from jax.experimental import pallas as pl
from jax.experimental.pallas import tpu as pltpu
from jax.experimental.pallas import tpu_sc as plsc

B, N, D = 8, 1250, 128
E = 40000
L = 4
Np = 1280
NC, NS = 2, 16
EP = E // NS
ROWS = 20
EPP = ROWS * 128
NPNP = Np * Np
STRIPE = NPNP // NS
ZCH = STRIPE // 8
GPC = B // NC
CB = Np // 128
CBSZ = Np * 128


def _sc_body(dst_hbm, src_hbm, vals_hbm, nvals_hbm, out_hbm,
             shared, srcv, dstv, idxb, valb, nvalb, zbuf, sem1, sem2, sem3):
  c = lax.axis_index("c")
  s = lax.axis_index("s")

  pltpu.sync_copy(vals_hbm, valb)
  pltpu.sync_copy(nvals_hbm, nvalb)

  def _z(i, _):
    zbuf[pl.ds(i * 16, 16)] = jnp.zeros((16,), jnp.float32)
    return 0
  lax.fori_loop(0, ZCH // 16, _z, 0)
  soff = pl.multiple_of(s * STRIPE, 256)
  for q in range(8):
    pltpu.sync_copy(zbuf, shared.at[pl.ds(soff + q * ZCH, ZCH)])
  plsc.subcore_barrier()

  def _graph(r, _):
    b = c * GPC + r
    cp1 = pltpu.async_copy(src_hbm.at[b, s], srcv, sem1)
    cp2 = pltpu.async_copy(dst_hbm.at[b, s], dstv, sem2)
    cp1.wait()
    cp2.wait()
    adds = []
    for j in range(ROWS):
      for k in range(8):
        sl = pl.ds(k * 16, 16)
        sv = srcv[j, sl]
        dv = dstv[j, sl]
        idxb[j, sl] = (lax.shift_right_logical(sv, 7) * CBSZ
                       + dv * 128 + lax.bitwise_and(sv, 127))
      adds.append(pltpu.async_copy(valb.at[j], shared.at[idxb.at[j]],
                                   sem3, add=True))
    for cp in adds:
      cp.wait()
    plsc.subcore_barrier()
    obase = pl.multiple_of(b * NPNP, 256)
    pltpu.sync_copy(shared.at[pl.ds(soff, STRIPE)],
                    out_hbm.at[pl.ds(obase + soff, STRIPE)])
    plsc.subcore_barrier()
    subs = [pltpu.async_copy(nvalb.at[j], shared.at[idxb.at[j]],
                             sem3, add=True) for j in range(ROWS)]
    for cp in subs:
      cp.wait()
    plsc.subcore_barrier()
    return 0

  lax.fori_loop(0, GPC, _graph, 0)


def _build_adjacency(dst4, src4, vals, nvals):
  mesh = plsc.VectorSubcoreMesh(core_axis_name="c", subcore_axis_name="s")
  f = pl.kernel(
      _sc_body,
      out_type=jax.ShapeDtypeStruct((B * NPNP,), jnp.float32),
      mesh=mesh,
      scratch_types=[
          pltpu.VMEM_SHARED((NPNP,), jnp.float32),
          pltpu.VMEM((ROWS, 128), jnp.int32),
          pltpu.VMEM((ROWS, 128), jnp.int32),
          pltpu.VMEM((ROWS, 128), jnp.int32),
          pltpu.VMEM((ROWS, 128), jnp.float32),
          pltpu.VMEM((ROWS, 128), jnp.float32),
          pltpu.VMEM((ZCH,), jnp.float32),
          pltpu.SemaphoreType.DMA,
          pltpu.SemaphoreType.DMA,
          pltpu.SemaphoreType.DMA,
      ],
  )
  return f(dst4, src4, vals, nvals)


def _tc_body(a_ref, x_ref, w_ref, b_ref, o_ref, an_ref):
  deg = jnp.ones((Np, 1), jnp.float32)
  for cb in range(CB):
    slab = a_ref[0, cb]
    an_ref[:, 128 * cb:128 * (cb + 1)] = slab
    deg = deg + jnp.sum(slab, axis=1, keepdims=True)
  dis = lax.rsqrt(deg)
  x = x_ref[0]
  for l in range(L):
    h = jnp.dot(x, w_ref[l], preferred_element_type=jnp.float32)
    hs = h * dis
    y = jnp.dot(an_ref[...], hs, preferred_element_type=jnp.float32) + hs
    x = jnp.tanh(y * dis + b_ref[l][None, :])
  o_ref[0] = x


def _gcn_stack(adj, x_pad, wst, bst):
  return pl.pallas_call(
      _tc_body,
      grid=(B,),
      in_specs=[
          pl.BlockSpec((1, CB, Np, 128), lambda b: (b, 0, 0, 0)),
          pl.BlockSpec((1, Np, D), lambda b: (b, 0, 0)),
          pl.BlockSpec((L, D, D), lambda b: (0, 0, 0)),
          pl.BlockSpec((L, D), lambda b: (0, 0)),
      ],
      out_specs=pl.BlockSpec((1, Np, D), lambda b: (b, 0, 0)),
      out_shape=jax.ShapeDtypeStruct((B, Np, D), jnp.float32),
      scratch_shapes=[pltpu.VMEM((Np, Np), jnp.float32)],
  )(adj, x_pad, wst, bst)


@jax.jit
def kernel(batch_node_tsr, edge_tsr_list, batch_last_node_idx_list,
           W0, b0, W1, b1, W2, b2, W3, b3):
  del batch_last_node_idx_list
  src = edge_tsr_list[:, 0, :].reshape(B, NS, EP)
  dst = edge_tsr_list[:, 1, :].reshape(B, NS, EP)
  pad = ((0, 0), (0, 0), (0, EPP - EP))
  src4 = jnp.pad(src, pad).reshape(B, NS, ROWS, 128)
  dst4 = jnp.pad(dst, pad).reshape(B, NS, ROWS, 128)
  valid = (jnp.arange(EPP) < EP).astype(jnp.float32).reshape(ROWS, 128)
  adj = _build_adjacency(dst4, src4, valid, -valid).reshape(B, CB, Np, 128)

  x_pad = jnp.pad(batch_node_tsr, ((0, 0), (0, Np - N), (0, 0)))
  wst = jnp.stack([W0, W1, W2, W3])
  bst = jnp.stack([b0, b1, b2, b3])
  out = _gcn_stack(adj, x_pad, wst, bst)
  return out[:, :N, :]


# speedup vs baseline: 83.0603x; 1.0061x over previous
import jax
import jax.numpy as jnp
from jax import lax
from jax.experimental import pallas as pl
from jax.experimental.pallas import tpu as pltpu
from jax.experimental.pallas import tpu_sc as plsc

B, N, D = 8, 1250, 128
E = 40000
L = 4
Np = 1280
NC, NS = 2, 16
EP = E // NS
ROWS = 20
EPP = ROWS * 128
NPNP = Np * Np
STRIPE = NPNP // NS
ZCH = STRIPE // 8
GPC = B // NC
CB = Np // 128
CBSZ = Np * 128


def _sc_body(dst_hbm, src_hbm, vals_hbm, nvals_hbm, out_hbm,
             shared, srcv, dstv, idxb, valb, nvalb, zbuf, sem1, sem2, sem3):
  c = lax.axis_index("c")
  s = lax.axis_index("s")

  pltpu.sync_copy(vals_hbm, valb)
  pltpu.sync_copy(nvals_hbm, nvalb)

  def _z(i, _):
    zbuf[pl.ds(i * 16, 16)] = jnp.zeros((16,), jnp.float32)
    return 0
  lax.fori_loop(0, ZCH // 16, _z, 0)
  soff = pl.multiple_of(s * STRIPE, 256)
  for q in range(8):
    pltpu.sync_copy(zbuf, shared.at[pl.ds(soff + q * ZCH, ZCH)])
  plsc.subcore_barrier()

  def _graph(r, _):
    b = c * GPC + r
    cp1 = pltpu.async_copy(src_hbm.at[b, s], srcv, sem1)
    cp2 = pltpu.async_copy(dst_hbm.at[b, s], dstv, sem2)
    cp1.wait()
    cp2.wait()
    adds = []
    for j in range(ROWS):
      for k in range(8):
        sl = pl.ds(k * 16, 16)
        sv = srcv[j, sl]
        dv = dstv[j, sl]
        idxb[j, sl] = (lax.shift_right_logical(sv, 7) * CBSZ
                       + dv * 128 + lax.bitwise_and(sv, 127))
      adds.append(pltpu.async_copy(valb.at[j], shared.at[idxb.at[j]],
                                   sem3, add=True))
    for cp in adds:
      cp.wait()
    plsc.subcore_barrier()
    obase = pl.multiple_of(b * NPNP, 256)
    pltpu.sync_copy(shared.at[pl.ds(soff, STRIPE)],
                    out_hbm.at[pl.ds(obase + soff, STRIPE)])
    plsc.subcore_barrier()
    subs = [pltpu.async_copy(nvalb.at[j], shared.at[idxb.at[j]],
                             sem3, add=True) for j in range(ROWS)]
    for cp in subs:
      cp.wait()
    plsc.subcore_barrier()
    return 0

  lax.fori_loop(0, GPC, _graph, 0)


def _build_adjacency(dst4, src4, vals, nvals):
  mesh = plsc.VectorSubcoreMesh(core_axis_name="c", subcore_axis_name="s")
  f = pl.kernel(
      _sc_body,
      out_type=jax.ShapeDtypeStruct((B * NPNP,), jnp.float32),
      mesh=mesh,
      scratch_types=[
          pltpu.VMEM_SHARED((NPNP,), jnp.float32),
          pltpu.VMEM((ROWS, 128), jnp.int32),
          pltpu.VMEM((ROWS, 128), jnp.int32),
          pltpu.VMEM((ROWS, 128), jnp.int32),
          pltpu.VMEM((ROWS, 128), jnp.float32),
          pltpu.VMEM((ROWS, 128), jnp.float32),
          pltpu.VMEM((ZCH,), jnp.float32),
          pltpu.SemaphoreType.DMA,
          pltpu.SemaphoreType.DMA,
          pltpu.SemaphoreType.DMA,
      ],
  )
  return f(dst4, src4, vals, nvals)


def _tc_body(a_ref, x_ref, w_ref, b_ref, o_ref, an_ref):
  deg = jnp.ones((Np, 1), jnp.float32)
  for cb in range(CB):
    slab = a_ref[0, cb]
    an_ref[:, 128 * cb:128 * (cb + 1)] = slab.astype(jnp.bfloat16)
    deg = deg + jnp.sum(slab, axis=1, keepdims=True)
  dis = lax.rsqrt(deg)
  x = x_ref[0]
  for l in range(L):
    h = jnp.dot(x, w_ref[l], preferred_element_type=jnp.float32)
    hs = h * dis
    y = jnp.dot(an_ref[...], hs.astype(jnp.bfloat16),
                preferred_element_type=jnp.float32) + hs
    x = jnp.tanh(y * dis + b_ref[l][None, :])
  o_ref[0] = x


def _gcn_stack(adj, x_pad, wst, bst):
  return pl.pallas_call(
      _tc_body,
      grid=(B,),
      in_specs=[
          pl.BlockSpec((1, CB, Np, 128), lambda b: (b, 0, 0, 0)),
          pl.BlockSpec((1, Np, D), lambda b: (b, 0, 0)),
          pl.BlockSpec((L, D, D), lambda b: (0, 0, 0)),
          pl.BlockSpec((L, D), lambda b: (0, 0)),
      ],
      out_specs=pl.BlockSpec((1, Np, D), lambda b: (b, 0, 0)),
      out_shape=jax.ShapeDtypeStruct((B, N, D), jnp.float32),
      scratch_shapes=[pltpu.VMEM((Np, Np), jnp.bfloat16)],
  )(adj, x_pad, wst, bst)


@jax.jit
def kernel(batch_node_tsr, edge_tsr_list, batch_last_node_idx_list,
           W0, b0, W1, b1, W2, b2, W3, b3):
  del batch_last_node_idx_list
  src = edge_tsr_list[:, 0, :].reshape(B, NS, EP)
  dst = edge_tsr_list[:, 1, :].reshape(B, NS, EP)
  pad = ((0, 0), (0, 0), (0, EPP - EP))
  src4 = jnp.pad(src, pad).reshape(B, NS, ROWS, 128)
  dst4 = jnp.pad(dst, pad).reshape(B, NS, ROWS, 128)
  valid = (jnp.arange(EPP) < EP).astype(jnp.float32).reshape(ROWS, 128)
  adj = _build_adjacency(dst4, src4, valid, -valid).reshape(B, CB, Np, 128)

  x_pad = jnp.pad(batch_node_tsr, ((0, 0), (0, Np - N), (0, 0)))
  wst = jnp.stack([W0, W1, W2, W3])
  bst = jnp.stack([b0, b1, b2, b3])
  return _gcn_stack(adj, x_pad, wst, bst)
